# Initial kernel scaffold; baseline (speedup 1.0000x reference)
import jax
import jax.numpy as jnp
from jax.experimental import pallas as pl


def kernel(logits, temperatures):
    raise NotImplementedError("write your pallas kernel here")


# speedup vs baseline: 4.7580x; 4.7580x over previous
import jax
import jax.numpy as jnp
from jax.experimental import pallas as pl
from jax.experimental.pallas import tpu as pltpu

B = 128
V = 100000
C = 2048
K = (V + C - 1) // C
RPS = 8

NEG_BIG = -3.0e38


def _stats_kernel(logits_ref, invt_ref, m_ref, s_ref):
    k = pl.program_id(0)
    x = logits_ref[...]
    invt = invt_ref[...]
    col = k * C + jax.lax.broadcasted_iota(jnp.int32, (B, C), 1)
    valid = col < V
    scaled = jnp.where(valid, x * invt, NEG_BIG)
    mk = jnp.max(scaled, axis=1, keepdims=True)
    e = jnp.where(valid, jnp.exp(scaled - mk), 0.0)
    sk = jnp.sum(e, axis=1, keepdims=True)
    lane = jax.lax.broadcasted_iota(jnp.int32, (B, K), 1)
    hit = lane == k
    m_ref[...] = jnp.where(hit, mk, m_ref[...])
    s_ref[...] = jnp.where(hit, sk, s_ref[...])


def _lane_shift_right(x, sh):
    r, w = x.shape
    return jnp.concatenate(
        [jnp.zeros((r, sh), x.dtype), x[:, :w - sh]], axis=1)


def _lane_cumsum(x):
    w = x.shape[1]
    sh = 1
    while sh < w:
        x = x + _lane_shift_right(x, sh)
        sh *= 2
    return x


def _merge_kernel(m_ref, s_ref, u_ref, kstar_ref, scal_ref):
    mloc = m_ref[...]
    sloc = s_ref[...]
    m = jnp.max(mloc, axis=1, keepdims=True)
    a = sloc * jnp.exp(mloc - m)
    prefix = _lane_cumsum(a)
    z = prefix[:, K - 1:K]
    t = u_ref[...] * z
    below = jnp.where(prefix < t, 1.0, 0.0)
    kraw = jnp.sum(below, axis=1, keepdims=True)
    kst = jnp.minimum(kraw, float(K - 1))
    kidx = jax.lax.broadcasted_iota(jnp.int32, (B, K), 1).astype(jnp.float32)
    pexc = jnp.sum(jnp.where(kidx < kst, a, 0.0), axis=1, keepdims=True)
    kstar_ref[...] = kst.astype(jnp.int32)
    scal_ref[...] = jnp.concatenate(
        [m, t, pexc, kst, jnp.zeros((B, 4), jnp.float32)], axis=1)


def _pick_kernel(kstar_pref, *refs):
    x_refs = refs[:RPS]
    scal_ref, invt_ref, out_ref = refs[RPS:]
    rows = jnp.concatenate(
        [x_refs[j][j:j + 1, :] for j in range(RPS)], axis=0)
    scal = scal_ref[...]
    m = scal[:, 0:1]
    t = scal[:, 1:2]
    pexc = scal[:, 2:3]
    kst = scal[:, 3:4]
    invt = invt_ref[...]
    col = kst * float(C) + jax.lax.broadcasted_iota(
        jnp.int32, (RPS, C), 1).astype(jnp.float32)
    valid = col < float(V)
    e = jnp.where(valid, jnp.exp(rows * invt - m), 0.0)
    prefix = pexc + _lane_cumsum(e)
    cnt = jnp.sum(jnp.where(prefix < t, 1.0, 0.0), axis=1, keepdims=True)
    sample = jnp.minimum(kst * float(C) + cnt, float(V - 1))
    out_ref[...] = jnp.broadcast_to(sample, (RPS, 128))


def kernel(logits, temperatures):
    u = jax.random.uniform(jax.random.key(42), (B, 1), dtype=jnp.float32)
    invt = (1.0 / temperatures).reshape(B, 1)

    m_chunk, s_chunk = pl.pallas_call(
        _stats_kernel,
        grid=(K,),
        in_specs=[
            pl.BlockSpec((B, C), lambda k: (0, k)),
            pl.BlockSpec((B, 1), lambda k: (0, 0)),
        ],
        out_specs=[
            pl.BlockSpec((B, K), lambda k: (0, 0)),
            pl.BlockSpec((B, K), lambda k: (0, 0)),
        ],
        out_shape=[
            jax.ShapeDtypeStruct((B, K), jnp.float32),
            jax.ShapeDtypeStruct((B, K), jnp.float32),
        ],
    )(logits, invt)

    kstar, scal = pl.pallas_call(
        _merge_kernel,
        out_shape=[
            jax.ShapeDtypeStruct((B, 1), jnp.int32),
            jax.ShapeDtypeStruct((B, 8), jnp.float32),
        ],
    )(m_chunk, s_chunk, u)

    in_specs = []
    for j in range(RPS):
        in_specs.append(pl.BlockSpec(
            (RPS, C), lambda i, ks, j=j: (i, ks[i * RPS + j])))
    in_specs.append(pl.BlockSpec((RPS, 8), lambda i, ks: (i, 0)))
    in_specs.append(pl.BlockSpec((RPS, 1), lambda i, ks: (i, 0)))

    out = pl.pallas_call(
        _pick_kernel,
        grid_spec=pltpu.PrefetchScalarGridSpec(
            num_scalar_prefetch=1,
            grid=(B // RPS,),
            in_specs=in_specs,
            out_specs=pl.BlockSpec((RPS, 128), lambda i, ks: (i, 0)),
        ),
        out_shape=jax.ShapeDtypeStruct((B, 128), jnp.float32),
    )(kstar.reshape(B), *([logits] * RPS), scal, invt)

    return out[:, 0].astype(jnp.int32)
